# baseline (speedup 1.0000x reference)
import functools

import jax
import jax.numpy as jnp
from jax import lax
from jax.experimental import pallas as pl
from jax.experimental.pallas import tpu as pltpu
from jax.experimental.pallas import tpu_sc as plsc

NC = 2
NS = 16
NW = NC * NS

V = 1000000
D = 64
LANE = 128
N_FULL_COLS = V // LANE
COLS_PER_W = N_FULL_COLS // NW
S = 4096
T = 200
NJ = S // LANE
NBLK = T * NJ
BLK_PER_W = NBLK // NW


def _mesh():
    return plsc.VectorSubcoreMesh(core_axis_name="c", subcore_axis_name="s")


def _wid():
    return lax.axis_index("s") * NC + lax.axis_index("c")


def _iota16():
    return lax.iota(jnp.int32, 16)


def _transpose_block(src, dst, n_pairs):
    i16 = _iota16()
    rows = [i16 + 16 * gg for gg in range(4)]

    def pbody(p, carry):
        for g in range(8):
            col = 2 * p + (1 if g >= 4 else 0)
            colv = jnp.full((16,), 0, jnp.int32) + col
            vals = plsc.load_gather(src, [rows[g % 4], colv])
            dst[p, pl.ds(16 * g, 16)] = vals
        return carry

    lax.fori_loop(0, n_pairs, pbody, 0)


@functools.partial(jax.jit)
def _format_table(xt):
    mesh = _mesh()

    def body(xt_hbm, tab_hbm, r0, r1, r2, r3, b0, b1, src_tail,
             rsem0, rsem1, rsem2, rsem3, bsem0, bsem1):
        rbuf = [r0, r1, r2, r3]
        rsem = [rsem0, rsem1, rsem2, rsem3]
        bbuf = [b0, b1]
        bsem = [bsem0, bsem1]
        w = _wid()
        c0 = w * COLS_PER_W

        def rd(k, p4):
            pltpu.async_copy(
                xt_hbm.at[:, pl.ds((c0 + k) * LANE, LANE)], rbuf[p4], rsem[p4]
            )

        def wr_desc(k, p2):
            return pltpu.make_async_copy(
                bbuf[p2], tab_hbm.at[pl.ds((c0 + k) * D, D)], bsem[p2]
            )

        rd(0, 0)
        rd(1, 1)
        rd(2, 2)

        def outer(k4, carry):
            for u in range(4):
                k = k4 * 4 + u
                p4 = u
                p2 = u % 2

                @pl.when(k + 3 < COLS_PER_W)
                def _():
                    rd(k + 3, (u + 3) % 4)

                pltpu.make_async_copy(
                    xt_hbm.at[:, pl.ds(0, LANE)], rbuf[p4], rsem[p4]
                ).wait()

                @pl.when(k >= 2)
                def _():
                    wr_desc(k - 2, p2).wait()

                _transpose_block(rbuf[p4], bbuf[p2], 64)
                wr_desc(k, p2).start()
            return carry

        lax.fori_loop(0, COLS_PER_W // 4, outer, 0)
        wr_desc(COLS_PER_W - 2, 0).wait()
        wr_desc(COLS_PER_W - 1, 1).wait()

        @pl.when(w < 4)
        def _():
            pltpu.sync_copy(xt_hbm.at[:, pl.ds((NW * COLS_PER_W + w) * LANE, LANE)], r0)
            _transpose_block(r0, b0, 64)
            pltpu.sync_copy(b0, tab_hbm.at[pl.ds((NW * COLS_PER_W + w) * D, D)])

        @pl.when(w == 4)
        def _():
            pltpu.sync_copy(xt_hbm.at[:, pl.ds(N_FULL_COLS * LANE, D)], src_tail)
            _transpose_block(src_tail, b0, 32)
            pltpu.sync_copy(
                b0.at[pl.ds(0, 32)], tab_hbm.at[pl.ds(N_FULL_COLS * D, 32)]
            )

    k = pl.kernel(
        body,
        out_type=jax.ShapeDtypeStruct((V // 2, LANE), jnp.float32),
        mesh=mesh,
        compiler_params=pltpu.CompilerParams(use_tc_tiling_on_sc=True, needs_layout_passes=False),
        scratch_types=(
            [pltpu.VMEM((D, LANE), jnp.float32) for _ in range(4)]
            + [pltpu.VMEM((D, LANE), jnp.float32) for _ in range(2)]
            + [pltpu.VMEM((D, D), jnp.float32)]
            + [pltpu.SemaphoreType.DMA for _ in range(6)]
        ),
    )
    return k(xt)


@functools.partial(jax.jit)
def _gather_out(tab, xi):
    mesh = _mesh()

    def body(tab_hbm, xi_hbm, out_hbm, xiv, pidx, g0, g1, g2, g3, o0, o1,
             gsem0, gsem1, gsem2, gsem3, osem0, osem1):
        gbuf = [g0, g1, g2, g3]
        gsem = [gsem0, gsem1, gsem2, gsem3]
        obuf = [o0, o1]
        osem = [osem0, osem1]
        w = _wid()
        b0 = w * BLK_PER_W
        i16 = _iota16()
        rows = [i16 + 16 * gg for gg in range(8)]

        pltpu.sync_copy(xi_hbm.at[pl.ds(b0, BLK_PER_W)], xiv)

        def prep_and_fire(k, ring):
            for g in range(8):
                vv = xiv[k, pl.ds(16 * g, 16)]
                pidx[ring, pl.ds(16 * g, 16)] = lax.shift_right_logical(vv, 1)
            pltpu.async_copy(tab_hbm.at[pidx.at[ring]], gbuf[ring], gsem[ring])

        def owrite_desc(k, p2):
            b = b0 + k
            t = b // NJ
            j = lax.rem(b, NJ)
            return pltpu.make_async_copy(
                obuf[p2], out_hbm.at[t, :, pl.ds(j * LANE, LANE)], osem[p2]
            )

        def transpose_out(k, p4, p2):
            src = gbuf[p4]
            dst = obuf[p2]
            colb = []
            for g in range(8):
                vv = xiv[k, pl.ds(16 * g, 16)]
                colb.append(lax.shift_left(jnp.bitwise_and(vv, 1), 6))

            def fbody(f, carry):
                for g in range(8):
                    vals = plsc.load_gather(src, [rows[g], colb[g] + f])
                    dst[f, pl.ds(16 * g, 16)] = vals
                return carry

            lax.fori_loop(0, D, fbody, 0)

        prep_and_fire(0, 0)
        prep_and_fire(1, 1)
        prep_and_fire(2, 2)

        def outer(k4, carry):
            for u in range(4):
                k = k4 * 4 + u
                p4 = u
                p2 = u % 2

                @pl.when(k + 3 < BLK_PER_W)
                def _():
                    prep_and_fire(k + 3, (u + 3) % 4)

                pltpu.make_async_copy(
                    tab_hbm.at[pl.ds(0, LANE)], gbuf[p4], gsem[p4]
                ).wait()

                @pl.when(k >= 2)
                def _():
                    owrite_desc(k - 2, p2).wait()

                transpose_out(k, p4, p2)
                owrite_desc(k, p2).start()
            return carry

        lax.fori_loop(0, BLK_PER_W // 4, outer, 0)
        owrite_desc(BLK_PER_W - 2, 0).wait()
        owrite_desc(BLK_PER_W - 1, 1).wait()

    k = pl.kernel(
        body,
        out_type=jax.ShapeDtypeStruct((T, D, S), jnp.float32),
        mesh=mesh,
        compiler_params=pltpu.CompilerParams(use_tc_tiling_on_sc=True, needs_layout_passes=False),
        scratch_types=(
            [pltpu.VMEM((BLK_PER_W, LANE), jnp.int32)]
            + [pltpu.VMEM((4, LANE), jnp.int32)]
            + [pltpu.VMEM((LANE, LANE), jnp.float32) for _ in range(4)]
            + [pltpu.VMEM((D, LANE), jnp.float32) for _ in range(2)]
            + [pltpu.SemaphoreType.DMA for _ in range(6)]
        ),
    )
    return k(tab, xi)


def kernel(x, emb_table):
    xt = emb_table.T
    xi = x.astype(jnp.int32).T.reshape(T, NJ, LANE).reshape(NBLK, LANE)
    tab = _format_table(xt)
    out_t = _gather_out(tab, xi)
    return jnp.transpose(out_t, (2, 0, 1))
